# baseline (speedup 1.0000x reference)
import functools

import jax
import jax.numpy as jnp
from jax import lax
from jax.experimental import pallas as pl
from jax.experimental.pallas import tpu as pltpu
from jax.experimental.pallas import tpu_sc as plsc

_TOPK = 64
_B = 128
_H = 32768
_NV = _H // 16
_NGRP = 16
_GV = _NV // _NGRP
_NCHUNK = _NGRP * 16
_CAND_MAX = _H
_INT_MIN = -2147483648
_INT_MAX = 2147483647


def _key16(v):
    u = lax.bitcast_convert_type(v, jnp.int32)
    return jnp.where(u >= 0, u, jnp.int32(_INT_MIN) - u)


def _inv_key16(k):
    u = jnp.where(k > 0, k, jnp.int32(_INT_MIN) - k)
    return lax.bitcast_convert_type(u, jnp.float32)


def _shuffle(v, perm):
    dn = lax.GatherDimensionNumbers(
        offset_dims=(), collapsed_slice_dims=(0,), start_index_map=(0,))
    return lax.gather(v, perm.reshape(16, 1), dn, (1,),
                      mode=lax.GatherScatterMode.PROMISE_IN_BOUNDS)


def _lanesum(c):
    io = lax.iota(jnp.int32, 16)
    for k in (1, 2, 4, 8):
        c = c + _shuffle(c, io ^ k)
    return c


def _popcnt(mask):
    return _lanesum(jnp.where(mask, jnp.full((16,), 1, jnp.int32),
                               jnp.full((16,), 0, jnp.int32)))


def _count_ge(buf, nv, mid):
    def body(i, c):
        k = _key16(buf[pl.ds(i * 16, 16)])
        return c + jnp.where(k >= mid, jnp.full((16,), 1, jnp.int32),
                             jnp.full((16,), 0, jnp.int32))

    return _lanesum(lax.fori_loop(0, nv, body, jnp.zeros((16,), jnp.int32)))


def _count_ge2(buf, nv2, mid):
    def body(i, cs):
        c0, c1 = cs
        k0 = _key16(buf[pl.ds(i * 32, 16)])
        k1 = _key16(buf[pl.ds(i * 32 + 16, 16)])
        one = jnp.full((16,), 1, jnp.int32)
        zero = jnp.full((16,), 0, jnp.int32)
        return (c0 + jnp.where(k0 >= mid, one, zero),
                c1 + jnp.where(k1 >= mid, one, zero))

    z = jnp.zeros((16,), jnp.int32)
    c0, c1 = lax.fori_loop(0, nv2, body, (z, z))
    return _lanesum(c0 + c1)


def _bisect_ge(buf, nv, lo, hi, target):
    def body(_, s):
        lo, hi = s
        mid = (lo >> 1) + (hi >> 1) + (lo & hi & 1) + ((lo ^ hi) & 1)
        p = _count_ge(buf, nv, mid) >= target
        return jnp.where(p, mid, lo), jnp.where(p, hi, mid - 1)

    return lax.fori_loop(0, 32, body, (lo, hi))[0]


def _splat(x):
    return jnp.full((16,), x, jnp.int32)


def _sc_body(x_hbm, out_hbm, xv, mv, cand):
    wid = lax.axis_index("s") * 2 + lax.axis_index("c")

    def row_body(j, _unused):
        base = (wid * 4 + j) * _H
        pltpu.sync_copy(x_hbm.at[pl.ds(base, _H)], xv)

        def grp(g, _):
            def inner(t, accs):
                a0, a1 = accs
                b = (g * _GV + t * 8) * 16
                v = [xv[pl.ds(b + k * 16, 16)] for k in range(8)]
                m01 = jnp.maximum(v[0], v[1])
                m23 = jnp.maximum(v[2], v[3])
                m45 = jnp.maximum(v[4], v[5])
                m67 = jnp.maximum(v[6], v[7])
                return (jnp.maximum(a0, jnp.maximum(m01, m23)),
                        jnp.maximum(a1, jnp.maximum(m45, m67)))

            ninf = jnp.full((16,), -jnp.inf, jnp.float32)
            a0, a1 = lax.fori_loop(0, _GV // 8, inner, (ninf, ninf))
            mv[pl.ds(g * 16, 16)] = jnp.maximum(a0, a1)
            return 0

        lax.fori_loop(0, _NGRP, grp, 0)

        mxv = mv[pl.ds(0, 16)]
        for g in range(1, _NGRP):
            mxv = jnp.maximum(mxv, mv[pl.ds(g * 16, 16)])
        io0 = lax.iota(jnp.int32, 16)
        for k in (1, 2, 4, 8):
            mxv = jnp.maximum(mxv, _shuffle(mxv, io0 ^ k))
        maxk = _key16(mxv)

        m65 = _bisect_ge(mv, _NGRP, _splat(_INT_MIN), maxk,
                         _splat(_TOPK + 1))
        m65f = _inv_key16(m65)

        def scan(i, ns):
            b = i * 8 * 16
            v = [xv[pl.ds(b + k * 16, 16)] for k in range(8)]
            m01 = jnp.maximum(v[0], v[1])
            m23 = jnp.maximum(v[2], v[3])
            m45 = jnp.maximum(v[4], v[5])
            m67 = jnp.maximum(v[6], v[7])
            mx = jnp.maximum(jnp.maximum(m01, m23), jnp.maximum(m45, m67))
            a = _popcnt(mx >= m65f)[0] > 0

            def hit(n):
                for k in range(8):
                    v = xv[pl.ds(b + k * 16, 16)]
                    m = v >= m65f
                    cand[pl.ds(n * 16, 16)] = jnp.where(
                        m, v, jnp.full((16,), -jnp.inf, jnp.float32))
                    n = n + jnp.where(_popcnt(m)[0] > 0,
                                      jnp.int32(1), jnp.int32(0))
                return n

            return lax.cond(a, hit, lambda n: n, ns)

        nvc = lax.fori_loop(0, _NV // 8, scan, jnp.int32(0))

        tkey = _bisect_ge(cand, nvc, m65, maxk, _splat(_TOPK))
        in_range = tkey < _INT_MAX
        c_gt = jnp.where(
            in_range,
            _count_ge(cand, nvc, jnp.where(in_range, tkey + 1, tkey)),
            _splat(0))
        need_eq = _splat(_TOPK) - c_gt
        tf = _inv_key16(tkey)

        def eqcount(i, c):
            kc = _key16(cand[pl.ds(i * 16, 16)])
            return c + jnp.where(kc == tkey, jnp.full((16,), 1, jnp.int32),
                                 jnp.full((16,), 0, jnp.int32))

        ce = _lanesum(
            lax.fori_loop(0, nvc, eqcount, jnp.zeros((16,), jnp.int32)))

        def out_scan(i, _):
            b = i * 8 * 16
            for k in range(8):
                v = xv[pl.ds(b + k * 16, 16)]
                xv[pl.ds(b + k * 16, 16)] = jnp.where(
                    v >= tf, v, jnp.float32(0.0))
            return 0

        lax.fori_loop(0, _NV // 8, out_scan, 0)

        excess = (ce - need_eq)[0]
        nfix = jnp.where(excess > 0, jnp.int32(_NV), jnp.int32(0))
        io = lax.iota(jnp.int32, 16)

        def fixscan(i, bv):
            v = xv[pl.ds(i * 16, 16)]
            meqi = jnp.where(v == tf, _splat(1), _splat(0))
            pre = meqi
            for k in (1, 2, 4, 8):
                shifted = _shuffle(pre, jnp.maximum(io - k, _splat(0)))
                pre = pre + jnp.where(io >= k, shifted, _splat(0))
            rank_ex = pre - meqi
            zm = meqi * jnp.where(rank_ex >= bv, _splat(1), _splat(0))
            xv[pl.ds(i * 16, 16)] = jnp.where(zm > 0, jnp.float32(0.0), v)
            return bv - _lanesum(meqi)

        lax.fori_loop(0, nfix, fixscan, need_eq)

        pltpu.sync_copy(xv, out_hbm.at[pl.ds(base, _H)])
        return 0

    lax.fori_loop(0, _B // 32, row_body, 0)


@functools.partial(
    pl.kernel,
    mesh=plsc.VectorSubcoreMesh(core_axis_name="c", subcore_axis_name="s"),
    out_type=jax.ShapeDtypeStruct((_B * _H,), jnp.float32),
    scratch_types=[
        pltpu.VMEM((_H,), jnp.float32),
        pltpu.VMEM((_NCHUNK,), jnp.float32),
        pltpu.VMEM((_CAND_MAX,), jnp.float32),
    ],
)
def _sc_topk(x_hbm, out_hbm, xv, mv, cand):
    _sc_body(x_hbm, out_hbm, xv, mv, cand)


def kernel(hidden_preactivation_BH):
    b, h = hidden_preactivation_BH.shape
    flat = hidden_preactivation_BH.reshape((b * h,))
    return _sc_topk(flat).reshape((b, h))


# speedup vs baseline: 1.1247x; 1.1247x over previous
import functools

import jax
import jax.numpy as jnp
from jax import lax
from jax.experimental import pallas as pl
from jax.experimental.pallas import tpu as pltpu
from jax.experimental.pallas import tpu_sc as plsc

_TOPK = 64
_B = 128
_SC_ROWS = 64
_RPW = _SC_ROWS // 32
_H = 32768
_NV = _H // 16
_NGRP = 16
_GV = _NV // _NGRP
_NCHUNK = _NGRP * 16
_CAND_MAX = _H
_INT_MIN = -2147483648
_INT_MAX = 2147483647


def _key16(v):
    u = lax.bitcast_convert_type(v, jnp.int32)
    return jnp.where(u >= 0, u, jnp.int32(_INT_MIN) - u)


def _inv_key16(k):
    u = jnp.where(k > 0, k, jnp.int32(_INT_MIN) - k)
    return lax.bitcast_convert_type(u, jnp.float32)


def _shuffle(v, perm):
    dn = lax.GatherDimensionNumbers(
        offset_dims=(), collapsed_slice_dims=(0,), start_index_map=(0,))
    return lax.gather(v, perm.reshape(16, 1), dn, (1,),
                      mode=lax.GatherScatterMode.PROMISE_IN_BOUNDS)


def _lanesum(c):
    io = lax.iota(jnp.int32, 16)
    for k in (1, 2, 4, 8):
        c = c + _shuffle(c, io ^ k)
    return c


def _popcnt(mask):
    return _lanesum(jnp.where(mask, jnp.full((16,), 1, jnp.int32),
                               jnp.full((16,), 0, jnp.int32)))


def _count_ge(buf, nv, mid):
    def body(i, c):
        k = _key16(buf[pl.ds(i * 16, 16)])
        return c + jnp.where(k >= mid, jnp.full((16,), 1, jnp.int32),
                             jnp.full((16,), 0, jnp.int32))

    return _lanesum(lax.fori_loop(0, nv, body, jnp.zeros((16,), jnp.int32)))


def _count_ge2(buf, nv2, mid):
    def body(i, cs):
        c0, c1 = cs
        k0 = _key16(buf[pl.ds(i * 32, 16)])
        k1 = _key16(buf[pl.ds(i * 32 + 16, 16)])
        one = jnp.full((16,), 1, jnp.int32)
        zero = jnp.full((16,), 0, jnp.int32)
        return (c0 + jnp.where(k0 >= mid, one, zero),
                c1 + jnp.where(k1 >= mid, one, zero))

    z = jnp.zeros((16,), jnp.int32)
    c0, c1 = lax.fori_loop(0, nv2, body, (z, z))
    return _lanesum(c0 + c1)


def _bisect_ge(buf, nv, lo, hi, target):
    def body(_, s):
        lo, hi = s
        mid = (lo >> 1) + (hi >> 1) + (lo & hi & 1) + ((lo ^ hi) & 1)
        p = _count_ge(buf, nv, mid) >= target
        return jnp.where(p, mid, lo), jnp.where(p, hi, mid - 1)

    return lax.fori_loop(0, 32, body, (lo, hi))[0]


def _splat(x):
    return jnp.full((16,), x, jnp.int32)


def _sc_body(x_hbm, out_hbm, xv, mv, cand):
    wid = lax.axis_index("s") * 2 + lax.axis_index("c")

    def row_body(j, _unused):
        base = (wid * _RPW + j) * _H
        pltpu.sync_copy(x_hbm.at[pl.ds(base, _H)], xv)

        def grp(g, _):
            def inner(t, accs):
                a0, a1 = accs
                b = (g * _GV + t * 8) * 16
                v = [xv[pl.ds(b + k * 16, 16)] for k in range(8)]
                m01 = jnp.maximum(v[0], v[1])
                m23 = jnp.maximum(v[2], v[3])
                m45 = jnp.maximum(v[4], v[5])
                m67 = jnp.maximum(v[6], v[7])
                return (jnp.maximum(a0, jnp.maximum(m01, m23)),
                        jnp.maximum(a1, jnp.maximum(m45, m67)))

            ninf = jnp.full((16,), -jnp.inf, jnp.float32)
            a0, a1 = lax.fori_loop(0, _GV // 8, inner, (ninf, ninf))
            mv[pl.ds(g * 16, 16)] = jnp.maximum(a0, a1)
            return 0

        lax.fori_loop(0, _NGRP, grp, 0)

        mxv = mv[pl.ds(0, 16)]
        for g in range(1, _NGRP):
            mxv = jnp.maximum(mxv, mv[pl.ds(g * 16, 16)])
        io0 = lax.iota(jnp.int32, 16)
        for k in (1, 2, 4, 8):
            mxv = jnp.maximum(mxv, _shuffle(mxv, io0 ^ k))
        maxk = _key16(mxv)

        m65 = _bisect_ge(mv, _NGRP, _splat(_INT_MIN), maxk,
                         _splat(_TOPK + 1))
        m65f = _inv_key16(m65)

        def scan(i, ns):
            b = i * 8 * 16
            v = [xv[pl.ds(b + k * 16, 16)] for k in range(8)]
            m01 = jnp.maximum(v[0], v[1])
            m23 = jnp.maximum(v[2], v[3])
            m45 = jnp.maximum(v[4], v[5])
            m67 = jnp.maximum(v[6], v[7])
            mx = jnp.maximum(jnp.maximum(m01, m23), jnp.maximum(m45, m67))
            a = _popcnt(mx >= m65f)[0] > 0

            def hit(n):
                for k in range(8):
                    v = xv[pl.ds(b + k * 16, 16)]
                    m = v >= m65f
                    cand[pl.ds(n * 16, 16)] = jnp.where(
                        m, v, jnp.full((16,), -jnp.inf, jnp.float32))
                    n = n + jnp.where(_popcnt(m)[0] > 0,
                                      jnp.int32(1), jnp.int32(0))
                return n

            return lax.cond(a, hit, lambda n: n, ns)

        nvc = lax.fori_loop(0, _NV // 8, scan, jnp.int32(0))

        tkey = _bisect_ge(cand, nvc, m65, maxk, _splat(_TOPK))
        in_range = tkey < _INT_MAX
        c_gt = jnp.where(
            in_range,
            _count_ge(cand, nvc, jnp.where(in_range, tkey + 1, tkey)),
            _splat(0))
        need_eq = _splat(_TOPK) - c_gt
        tf = _inv_key16(tkey)

        def eqcount(i, c):
            kc = _key16(cand[pl.ds(i * 16, 16)])
            return c + jnp.where(kc == tkey, jnp.full((16,), 1, jnp.int32),
                                 jnp.full((16,), 0, jnp.int32))

        ce = _lanesum(
            lax.fori_loop(0, nvc, eqcount, jnp.zeros((16,), jnp.int32)))

        def out_scan(i, _):
            b = i * 8 * 16
            for k in range(8):
                v = xv[pl.ds(b + k * 16, 16)]
                xv[pl.ds(b + k * 16, 16)] = jnp.where(
                    v >= tf, v, jnp.float32(0.0))
            return 0

        lax.fori_loop(0, _NV // 8, out_scan, 0)

        excess = (ce - need_eq)[0]
        nfix = jnp.where(excess > 0, jnp.int32(_NV), jnp.int32(0))
        io = lax.iota(jnp.int32, 16)

        def fixscan(i, bv):
            v = xv[pl.ds(i * 16, 16)]
            meqi = jnp.where(v == tf, _splat(1), _splat(0))
            pre = meqi
            for k in (1, 2, 4, 8):
                shifted = _shuffle(pre, jnp.maximum(io - k, _splat(0)))
                pre = pre + jnp.where(io >= k, shifted, _splat(0))
            rank_ex = pre - meqi
            zm = meqi * jnp.where(rank_ex >= bv, _splat(1), _splat(0))
            xv[pl.ds(i * 16, 16)] = jnp.where(zm > 0, jnp.float32(0.0), v)
            return bv - _lanesum(meqi)

        lax.fori_loop(0, nfix, fixscan, need_eq)

        pltpu.sync_copy(xv, out_hbm.at[pl.ds(base, _H)])
        return 0

    lax.fori_loop(0, _RPW, row_body, 0)


@functools.partial(
    pl.kernel,
    mesh=plsc.VectorSubcoreMesh(core_axis_name="c", subcore_axis_name="s"),
    out_type=jax.ShapeDtypeStruct((_SC_ROWS * _H,), jnp.float32),
    scratch_types=[
        pltpu.VMEM((_H,), jnp.float32),
        pltpu.VMEM((_NCHUNK,), jnp.float32),
        pltpu.VMEM((_CAND_MAX,), jnp.float32),
    ],
)
def _sc_topk(x_hbm, out_hbm, xv, mv, cand):
    _sc_body(x_hbm, out_hbm, xv, mv, cand)


_ROWS_PER_BLOCK = 8


def _tc_body(x_ref, o_ref):
    x = x_ref[...]
    u = lax.bitcast_convert_type(x, jnp.int32)
    key = jnp.where(u >= 0, u, jnp.int32(_INT_MIN) - u)
    lo = jnp.full((_ROWS_PER_BLOCK, 1), _INT_MIN, jnp.int32)
    hi = jnp.full((_ROWS_PER_BLOCK, 1), _INT_MAX, jnp.int32)

    def body(_, lohi):
        lo, hi = lohi
        mid = (lo >> 1) + (hi >> 1) + (lo & hi & 1) + ((lo ^ hi) & 1)
        cnt = jnp.sum((key >= mid).astype(jnp.int32), axis=1, keepdims=True)
        p = cnt >= _TOPK
        return jnp.where(p, mid, lo), jnp.where(p, hi, mid - 1)

    lo, hi = jax.lax.fori_loop(0, 32, body, (lo, hi))
    t = lo
    gt = key > t
    eq = key == t
    c_gt = jnp.sum(gt.astype(jnp.int32), axis=1, keepdims=True)
    need = _TOPK - c_gt
    eq_i = eq.astype(jnp.int32)
    ssum = eq_i
    shift = 1
    h = x.shape[1]
    while shift < h:
        shifted = jnp.concatenate(
            [jnp.zeros((_ROWS_PER_BLOCK, shift), jnp.int32),
             ssum[:, :-shift]], axis=1)
        ssum = ssum + shifted
        shift *= 2
    eq_rank = ssum - eq_i
    take = gt | (eq & (eq_rank < need))
    o_ref[...] = jnp.where(take, x, jnp.float32(0.0))


def _tc_topk(x):
    b, h = x.shape
    return pl.pallas_call(
        _tc_body,
        grid=(b // _ROWS_PER_BLOCK,),
        in_specs=[pl.BlockSpec((_ROWS_PER_BLOCK, h), lambda i: (i, 0))],
        out_specs=pl.BlockSpec((_ROWS_PER_BLOCK, h), lambda i: (i, 0)),
        out_shape=jax.ShapeDtypeStruct((b, h), jnp.float32),
    )(x)


def kernel(hidden_preactivation_BH):
    b, h = hidden_preactivation_BH.shape
    sc_in = hidden_preactivation_BH[:_SC_ROWS].reshape((_SC_ROWS * h,))
    sc_out = _sc_topk(sc_in).reshape((_SC_ROWS, h))
    tc_out = _tc_topk(hidden_preactivation_BH[_SC_ROWS:])
    return jnp.concatenate([sc_out, tc_out], axis=0)
